# baseline (speedup 1.0000x reference)
import functools

import jax
import jax.numpy as jnp
from jax import lax
from jax.experimental import pallas as pl
from jax.experimental.pallas import tpu as pltpu
from jax.experimental.pallas import tpu_sc as plsc

NC = 2
NS = 16
NW = NC * NS
LANES = 16
NB = 3


def _build(B, T, V, D):
    N = B * T
    assert B % NW == 0
    CPW = B // NW
    RPW = CPW * T
    assert CPW >= NB + 2
    JD = D // LANES

    mesh = plsc.VectorSubcoreMesh(
        core_axis_name="c", subcore_axis_name="s", num_cores=NC, num_subcores=NS
    )

    @functools.partial(
        pl.kernel,
        out_type=jax.ShapeDtypeStruct((B, T, D), jnp.float32),
        mesh=mesh,
        scratch_types=[
            pltpu.VMEM((RPW,), jnp.int32),
            pltpu.VMEM((T, D), jnp.float32),
            pltpu.VMEM((NB, T, D), jnp.float32),
            pltpu.SemaphoreType.DMA((NB,)),
            pltpu.SemaphoreType.DMA((NB,)),
        ],
    )
    def emb(x_hbm, tok_hbm, pos_hbm, out_hbm, idx_v, pos_v, buf, gsem, ssem):
        wid = lax.axis_index("s") * NC + lax.axis_index("c")
        pltpu.sync_copy(x_hbm.at[pl.ds(wid * RPW, RPW)], idx_v)
        pltpu.sync_copy(pos_hbm, pos_v)
        rbase = wid * CPW

        def issue_gather(c):
            b = c % NB
            pltpu.async_copy(
                tok_hbm.at[idx_v.at[pl.ds(c * T, T)]], buf.at[b], gsem.at[b])

        def wait_gather(c):
            b = c % NB
            pltpu.make_async_copy(
                tok_hbm.at[idx_v.at[pl.ds(c * T, T)]], buf.at[b],
                gsem.at[b]).wait()

        def issue_store(c):
            b = c % NB
            pltpu.async_copy(buf.at[b], out_hbm.at[rbase + c], ssem.at[b])

        def wait_store(c):
            b = c % NB
            pltpu.make_async_copy(
                buf.at[b], out_hbm.at[rbase + c], ssem.at[b]).wait()

        def add_pos(c):
            b = c % NB

            @pl.loop(0, T)
            def _row(r):
                for j in range(JD):
                    sl = pl.ds(j * LANES, LANES)
                    buf[b, r, sl] += pos_v[r, sl]

        issue_gather(0)
        issue_gather(1)

        wait_gather(0)
        add_pos(0)
        issue_store(0)
        issue_gather(2)

        wait_gather(1)
        add_pos(1)
        issue_store(1)
        wait_store(0)
        issue_gather(3)

        @pl.loop(2, CPW - 2)
        def _body(c):
            wait_gather(c)
            add_pos(c)
            issue_store(c)
            wait_store(c - 1)
            issue_gather(c + 2)

        for c in (CPW - 2, CPW - 1):
            wait_gather(c)
            add_pos(c)
            issue_store(c)

        for c in (CPW - 3, CPW - 2, CPW - 1):
            wait_store(c)

    return emb


def kernel(x, token_table, pos_table):
    B, T = x.shape
    V, D = token_table.shape
    emb = _build(B, T, V, D)
    return emb(x.astype(jnp.int32).reshape(-1), token_table, pos_table)


# speedup vs baseline: 2.4108x; 2.4108x over previous
import functools

import jax
import jax.numpy as jnp
from jax import lax
from jax.experimental import pallas as pl
from jax.experimental.pallas import tpu as pltpu
from jax.experimental.pallas import tpu_sc as plsc

NC = 2
NS = 16
NW = NC * NS
LANES = 16
SEG = 104
POFF = (0, 96)


def _build(B, T, V, D):
    assert B % NW == 0
    RPW = B // NW
    assert RPW >= 3
    assert SEG + POFF[1] == T and POFF[1] % 8 == 0
    JD = D // LANES

    mesh = plsc.VectorSubcoreMesh(
        core_axis_name="c", subcore_axis_name="s", num_cores=NC, num_subcores=NS
    )

    @functools.partial(
        pl.kernel,
        out_type=jax.ShapeDtypeStruct((B, T, D), jnp.float32),
        mesh=mesh,
        scratch_types=[
            pltpu.VMEM((RPW, 2, SEG), jnp.int32),
            pltpu.VMEM((T, D), jnp.float32),
            pltpu.VMEM((2, SEG, D), jnp.float32),
            pltpu.VMEM((2, SEG, D), jnp.float32),
            pltpu.SemaphoreType.DMA,
            pltpu.SemaphoreType.DMA,
            pltpu.SemaphoreType.DMA,
            pltpu.SemaphoreType.DMA,
        ],
    )
    def emb(x_hbm, tok_hbm, pos_hbm, out_hbm, idx_v, pos_v, gbuf, sbuf,
            g0, g1, s0, s1):
        gsems = (g0, g1)
        ssems = (s0, s1)
        wid = lax.axis_index("s") * NC + lax.axis_index("c")
        pltpu.sync_copy(x_hbm.at[wid], idx_v)
        pltpu.sync_copy(pos_hbm, pos_v)
        rbase = wid * RPW

        def issue_gather(r, b):
            pltpu.async_copy(
                tok_hbm.at[idx_v.at[r, b]], gbuf.at[b], gsems[b])

        def wait_gather(r, b):
            pltpu.make_async_copy(
                tok_hbm.at[idx_v.at[r, b]], gbuf.at[b], gsems[b]).wait()

        def issue_store(r, b):
            pltpu.async_copy(
                sbuf.at[b], out_hbm.at[rbase + r, pl.ds(POFF[b], SEG)],
                ssems[b])

        def wait_store(r, b):
            pltpu.make_async_copy(
                sbuf.at[b], out_hbm.at[rbase + r, pl.ds(POFF[b], SEG)],
                ssems[b]).wait()

        def add_pos(b):
            @pl.loop(0, SEG)
            def _row(i):
                for j in range(JD):
                    sl = pl.ds(j * LANES, LANES)
                    sbuf[b, i, sl] = gbuf[b, i, sl] + pos_v[POFF[b] + i, sl]

        for b in range(2):
            issue_gather(0, b)

        for b in range(2):
            wait_gather(0, b)
            add_pos(b)
            issue_gather(1, b)
            issue_store(0, b)

        @pl.loop(1, RPW - 1)
        def _body(r):
            for b in range(2):
                wait_gather(r, b)
                wait_store(r - 1, b)
                add_pos(b)
                issue_gather(r + 1, b)
                issue_store(r, b)

        for b in range(2):
            wait_gather(RPW - 1, b)
            wait_store(RPW - 2, b)
            add_pos(b)
            issue_store(RPW - 1, b)

        for b in range(2):
            wait_store(RPW - 1, b)

    return emb


def kernel(x, token_table, pos_table):
    B, T = x.shape
    V, D = token_table.shape
    emb = _build(B, T, V, D)
    xi = x.astype(jnp.int32)
    segs = jnp.stack([xi[:, 0:SEG], xi[:, POFF[1]:T]], axis=1)
    x_seg = segs.reshape(NW, B // NW, 2, SEG)
    return emb(x_seg, token_table, pos_table)
